# baseline (speedup 1.0000x reference)
import functools

import jax
import jax.numpy as jnp
from jax.experimental import pallas as pl
from jax.experimental.pallas import tpu as pltpu

N = 2048
D = 1024
F = 2048
E = 8
BN = 1024


FC = 512


def _moe_body(x_ref, w1_ref, b1_ref, w2_ref, b2_ref, rt_ref, out_ref):
    e = pl.program_id(1)
    x = x_ref[...]
    y = None
    for c in range(F // FC):
        sl = slice(c * FC, (c + 1) * FC)
        h = jnp.dot(x, w1_ref[0][:, sl], preferred_element_type=jnp.float32)
        h = h + b1_ref[0][:, sl]
        h = 0.5 * h * (1.0 + jax.lax.erf(h * 0.7071067811865476))
        p = jnp.dot(h.astype(jnp.bfloat16), w2_ref[0][sl, :],
                    preferred_element_type=jnp.float32)
        y = p if y is None else y + p
    y = y + b2_ref[0]
    scale = rt_ref[0].reshape(BN, 1)
    contrib = y * scale

    @pl.when(e == 0)
    def _init():
        out_ref[...] = contrib

    @pl.when(e != 0)
    def _acc():
        out_ref[...] += contrib


@jax.jit
def kernel(x, routing_tensor, W1, b1, W2, b2):
    xb = x.astype(jnp.bfloat16)
    w1b = W1.astype(jnp.bfloat16)
    w2b = W2.astype(jnp.bfloat16)
    b1r = b1.reshape(E, 1, F)
    b2r = b2.reshape(E, 1, D)
    rt = routing_tensor.T.reshape(E, 1, N)

    grid = (N // BN, E)
    return pl.pallas_call(
        _moe_body,
        grid=grid,
        in_specs=[
            pl.BlockSpec((BN, D), lambda t, e: (t, 0)),
            pl.BlockSpec((1, D, F), lambda t, e: (e, 0, 0)),
            pl.BlockSpec((1, 1, F), lambda t, e: (e, 0, 0)),
            pl.BlockSpec((1, F, D), lambda t, e: (e, 0, 0)),
            pl.BlockSpec((1, 1, D), lambda t, e: (e, 0, 0)),
            pl.BlockSpec((1, 1, BN), lambda t, e: (e, 0, t)),
        ],
        out_specs=pl.BlockSpec((BN, D), lambda t, e: (t, 0)),
        out_shape=jax.ShapeDtypeStruct((N, D), jnp.float32),
        compiler_params=pltpu.CompilerParams(
            dimension_semantics=("parallel", "arbitrary"),
        ),
    )(xb, w1b, b1r, w2b, b2r, rt)


# speedup vs baseline: 1.4811x; 1.4811x over previous
import jax
import jax.numpy as jnp
from jax.experimental import pallas as pl
from jax.experimental.pallas import tpu as pltpu

N = 2048
D = 1024
F = 2048
E = 8
BN = 1024


def _moe_body(x_ref, w1_ref, b1_ref, w2_ref, b2_ref, rt_ref, out_ref):
    e = pl.program_id(1)
    h = jnp.dot(x_ref[...], w1_ref[0], preferred_element_type=jnp.float32)
    h = h + b1_ref[0]
    h = 0.5 * h * (1.0 + jax.lax.erf(h * 0.7071067811865476))
    y = jnp.dot(h, w2_ref[0], preferred_element_type=jnp.float32)
    y = y + b2_ref[0]
    scale = rt_ref[0].reshape(BN, 1)
    contrib = y * scale

    @pl.when(e == 0)
    def _init():
        out_ref[...] = contrib

    @pl.when(e != 0)
    def _acc():
        out_ref[...] += contrib


@jax.jit
def kernel(x, routing_tensor, W1, b1, W2, b2):
    b1r = b1.reshape(E, 1, F)
    b2r = b2.reshape(E, 1, D)
    rt = routing_tensor.T.reshape(E, 1, N)

    grid = (N // BN, E)
    return pl.pallas_call(
        _moe_body,
        grid=grid,
        in_specs=[
            pl.BlockSpec((BN, D), lambda t, e: (t, 0)),
            pl.BlockSpec((1, D, F), lambda t, e: (e, 0, 0)),
            pl.BlockSpec((1, 1, F), lambda t, e: (e, 0, 0)),
            pl.BlockSpec((1, F, D), lambda t, e: (e, 0, 0)),
            pl.BlockSpec((1, 1, D), lambda t, e: (e, 0, 0)),
            pl.BlockSpec((1, 1, BN), lambda t, e: (e, 0, t)),
        ],
        out_specs=pl.BlockSpec((BN, D), lambda t, e: (t, 0)),
        out_shape=jax.ShapeDtypeStruct((N, D), jnp.float32),
        compiler_params=pltpu.CompilerParams(
            dimension_semantics=("parallel", "arbitrary"),
        ),
    )(x, W1, b1r, W2, b2r, rt)
